# baseline (speedup 1.0000x reference)
import functools
import math

import jax
import jax.numpy as jnp
import numpy as np
from jax import lax
from jax.experimental import pallas as pl
from jax.experimental.pallas import tpu as pltpu
from jax.experimental.pallas import tpu_sc as plsc

TABLE_NUM = 16
T = 4096
IMG = 256
NB = 4
CH = 32
RES_MIN, RES_MAX = 16, 256
MLP_HID = 64
_SQ2 = math.sqrt(2.0)

_growth = math.exp((math.log(RES_MAX) - math.log(RES_MIN)) / (TABLE_NUM - 1))
_RES = [int(math.floor(RES_MIN * (_growth ** l))) for l in range(TABLE_NUM)]
_R1 = [r + 1 for r in _RES]
_HASH_K = np.uint32(2654435761)


def _interp_mat(res):
    p = np.arange(IMG, dtype=np.float64)
    pos = (p + 0.5) * res / IMG
    p0 = np.floor(pos).astype(np.int64)
    f = pos - p0
    w = np.zeros((IMG, res + 1), np.float32)
    w[np.arange(IMG), p0] = (1.0 - f).astype(np.float32)
    w[np.arange(IMG), p0 + 1] += f.astype(np.float32)
    return w


_WY = [_interp_mat(r) for r in _RES]
_WXT = [np.ascontiguousarray(w.T) for w in _WY]


def _hash_idx(res):
    yi = np.arange(res + 1, dtype=np.uint32)
    xi = np.arange(res + 1, dtype=np.uint32)
    h = (xi[None, :] ^ (yi[:, None] * _HASH_K)) & np.uint32(T - 1)
    return h.astype(np.int32).reshape(-1)


_N = [r1 * r1 for r1 in _R1]
_CHK = 8192
_W = [((n + 127) // 128) * 128 for n in _N]
_NCH = [(w + _CHK - 1) // _CHK for w in _W]
_M = []
for _l in range(TABLE_NUM):
    _m = max(_R1[_l], -(-_W[_l] // _R1[_l]))
    while (_m * _R1[_l]) % 128 != 0:
        _m += 1
    _M.append(_m)
_SEG = [_M[_l] * _R1[_l] for _l in range(TABLE_NUM)]
_GA = [l for l in range(TABLE_NUM) if _R1[l] <= 128]
_GB = [l for l in range(TABLE_NUM) if _R1[l] > 128]
_MOFF = {}
_o = 0
for _l in _GA:
    _MOFF[_l] = _o
    _o += _M[_l]
_MA = _o
_o = 0
for _l in _GB:
    _MOFF[_l] = _o
    _o += _M[_l]
_MB = _o
_OFF = []
_segs = []
_off = 0
for _l in range(TABLE_NUM):
    _OFF.append(_off)
    _seg = np.zeros((_W[_l],), np.int32)
    _seg[: _N[_l]] = _hash_idx(_RES[_l])
    _segs.append(_seg)
    _off += _W[_l]
_IDX_FLAT = np.concatenate(_segs)

_JOBS = []
for _l in range(TABLE_NUM):
    for _ci in range(_NCH[_l]):
        _k = min(_CHK, _W[_l] - _ci * _CHK)
        _JOBS.append((_l, _ci, _k))
_NW = 32
_ASSIGN = [j % _NW for j in range(len(_JOBS))]


def _lrelu(y):
    return jnp.where(y >= 0, y, 0.2 * y) * _SQ2


def _dot_t(x, w):
    return lax.dot_general(x, w, (((1,), (1,)), ((), ())),
                           preferred_element_type=jnp.float32)


def _dot(x, w):
    return lax.dot_general(x, w, (((1,), (0,)), ((), ())),
                           preferred_element_type=jnp.float32)


def _dot_hi(x, w):
    return lax.dot_general(x, w, (((1,), (0,)), ((), ())),
                           precision=lax.Precision.HIGHEST,
                           preferred_element_type=jnp.float32)


_UP_FIN = [32 * (2 ** i) for i in range(7)]


def _a_body(z, mw0, mb0, mw1, mb2_0, mw2, mb2_2, bt,
            uw0, ua0, uw1, ua1, uw2, ua2, uw3, ua3, uw4, ua4, uw5, ua5,
            uw6, ua6, mla0, mlw0, mla1, mlw1, mla2, mlw2,
            tabs_o, sc0_o, dm0_o, sc1_o, dm1_o, sc2_o, dm2_o):
    zv = z[...]
    s = _lrelu(_dot_t(zv, mw0[...] * (1.0 / math.sqrt(512.0))) + mb0[...][None, :])
    s = _lrelu(_dot_t(s, mw1[...] * (1.0 / math.sqrt(256.0))) + mb2_0[...][None, :])
    s = _lrelu(_dot_t(s, mw2[...] * (1.0 / math.sqrt(256.0))) + mb2_2[...][None, :])

    x = jnp.broadcast_to(bt[...][None], (NB, CH, 32)).reshape(NB * CH, 32)
    uws = [uw0, uw1, uw2, uw3, uw4, uw5, uw6]
    uas = [ua0, ua1, ua2, ua3, ua4, ua5, ua6]
    for i in range(7):
        fin = _UP_FIN[i]
        w = uws[i][...]
        scale = _dot_t(s, uas[i][...]) + 1.0
        ssq = scale * scale
        acc = jnp.zeros((NB, 2 * fin), jnp.float32)
        ckn = 256
        for c0 in range(0, fin, ckn):
            ck = min(ckn, fin - c0)
            wz = w[:, c0:c0 + ck]
            acc = acc + _dot_t(ssq[:, c0:c0 + ck], wz * wz)
        demod = lax.rsqrt(acc + 1e-8)
        xs = x * jnp.broadcast_to(scale[:, None, :], (NB, CH, fin)).reshape(NB * CH, fin)
        y = _dot_t(xs, w)
        y = y * jnp.broadcast_to(demod[:, None, :], (NB, CH, 2 * fin)).reshape(NB * CH, 2 * fin)
        x = _lrelu(y)
    tabs_o[...] = x.reshape(NB, CH, T)

    def modpair(a_ref, w_ref):
        sc = _dot_t(s, a_ref[...]) + 1.0
        wv = w_ref[...]
        dm = lax.rsqrt(_dot_t(sc * sc, wv * wv) + 1e-8)
        return sc, dm

    sc0, dm0 = modpair(mla0, mlw0)
    sc1, dm1 = modpair(mla1, mlw1)
    sc2, dm2 = modpair(mla2, mlw2)
    sc0_o[...] = sc0
    dm0_o[...] = dm0
    sc1_o[...] = sc1
    dm1_o[...] = dm1
    sc2_o[...] = sc2
    dm2_o[...] = dm2


def _stage_a(args):
    out_shape = (
        jax.ShapeDtypeStruct((NB, CH, T), jnp.float32),
        jax.ShapeDtypeStruct((NB, CH), jnp.float32),
        jax.ShapeDtypeStruct((NB, MLP_HID), jnp.float32),
        jax.ShapeDtypeStruct((NB, MLP_HID), jnp.float32),
        jax.ShapeDtypeStruct((NB, MLP_HID), jnp.float32),
        jax.ShapeDtypeStruct((NB, MLP_HID), jnp.float32),
        jax.ShapeDtypeStruct((NB, 3), jnp.float32),
    )
    return pl.pallas_call(_a_body, out_shape=out_shape)(*args)


def _sc_body(tabs, idx_hbm, *refs):
    outs = refs[:TABLE_NUM]
    tab_v = refs[TABLE_NUM:TABLE_NUM + 8]
    idx_v = refs[TABLE_NUM + 8]
    gbuf = refs[TABLE_NUM + 9]
    wid = lax.axis_index("s") * 2 + lax.axis_index("c")
    for j, (l, ci, kk) in enumerate(_JOBS):
        @pl.when(wid == _ASSIGN[j])
        def _(l=l, ci=ci, kk=kk):
            for b in range(NB):
                for c in range(2):
                    row = b * CH + 2 * l + c
                    pltpu.sync_copy(tabs.at[pl.ds(row * T, T)], tab_v[2 * b + c])
            pltpu.sync_copy(idx_hbm.at[pl.ds(_OFF[l] + ci * _CHK, kk)],
                            idx_v.at[pl.ds(0, kk)])

            def body(i, carry):
                iv0 = idx_v[pl.ds(i * 32, 16)]
                iv1 = idx_v[pl.ds(i * 32 + 16, 16)]
                for ch in range(8):
                    g0 = plsc.load_gather(tab_v[ch], [iv0])
                    g1 = plsc.load_gather(tab_v[ch], [iv1])
                    gbuf[pl.ds(ch * kk + i * 32, 16)] = g0
                    gbuf[pl.ds(ch * kk + i * 32 + 16, 16)] = g1
                return carry

            lax.fori_loop(0, kk // 32, body, 0)
            for ch in range(8):
                pltpu.sync_copy(
                    gbuf.at[pl.ds(ch * kk, kk)],
                    outs[l].at[pl.ds(ch * _SEG[l] + ci * _CHK, kk)])


def _stage_b(tabs, idx):
    fn = pl.kernel(
        _sc_body,
        out_type=[jax.ShapeDtypeStruct((8 * _SEG[l],), jnp.float32)
                  for l in range(TABLE_NUM)],
        mesh=plsc.VectorSubcoreMesh(core_axis_name="c", subcore_axis_name="s"),
        compiler_params=pltpu.CompilerParams(needs_layout_passes=False),
        scratch_types=(
            [pltpu.VMEM((T,), jnp.float32) for _ in range(8)]
            + [pltpu.VMEM((_CHK,), jnp.int32)]
            + [pltpu.VMEM((8 * _CHK,), jnp.float32)]
        ),
    )
    return fn(tabs, idx)


def _c1_body(*refs):
    ga, gb = refs[0], refs[1]
    wys = refs[2:2 + TABLE_NUM]
    wxts = refs[2 + TABLE_NUM:2 + 2 * TABLE_NUM]
    out = refs[2 + 2 * TABLE_NUM]
    for l in range(TABLE_NUM):
        wy = wys[l][...]
        wxt = wxts[l][...]
        gref = ga if l in _GA else gb
        mo = _MOFF[l]
        for c in range(2):
            g = gref[c, mo:mo + _R1[l], 0:_R1[l]]
            h = _dot_hi(g, wxt)
            f = _dot_hi(wy, h)
            out[0, 2 * l + c] = f


def _stage_c1(ga, gb):
    in_specs = (
        [pl.BlockSpec((2, _MA, 128), lambda b: (b, 0, 0)),
         pl.BlockSpec((2, _MB, 256), lambda b: (b, 0, 0))]
        + [pl.BlockSpec((IMG, _R1[l]), lambda b: (0, 0))
           for l in range(TABLE_NUM)]
        + [pl.BlockSpec((_R1[l], IMG), lambda b: (0, 0))
           for l in range(TABLE_NUM)]
    )
    fn = pl.pallas_call(
        _c1_body,
        grid=(NB,),
        in_specs=in_specs,
        out_specs=pl.BlockSpec((1, CH, IMG, IMG), lambda b: (b, 0, 0, 0)),
        out_shape=jax.ShapeDtypeStruct((NB, CH, IMG, IMG), jnp.float32),
    )
    return fn(ga, gb, *[jnp.asarray(w) for w in _WY],
              *[jnp.asarray(w) for w in _WXT])


_LB = 8192


def _c2_body(feat, sc0, dm0, sc1, dm1, sc2, dm2, w0, b0, w1, b1, w2, b2, out):
    x = feat[...].reshape(NB * CH, _LB)
    xs = x * sc0[...][:, None]
    h = _dot(w0[...], xs) * dm0[...][:, None] + b0[...][:, None]
    h = _lrelu(h)
    hs = h * sc1[...][:, None]
    h = _dot(w1[...], hs) * dm1[...][:, None] + b1[...][:, None]
    h = _lrelu(h)
    hs = h * sc2[...][:, None]
    out[...] = _dot(w2[...], hs) * dm2[...][:, None] + b2[...][:, None]


def _stage_c2(featv, sc0, dm0, sc1, dm1, sc2, dm2, w0, b0, w1, b1, w2, b2):
    nblk = IMG * IMG // _LB
    eye = jnp.eye(NB, dtype=jnp.float32)
    w0b = jnp.kron(eye, w0)
    w1b = jnp.kron(eye, w1)
    w2b = jnp.kron(eye, w2)
    vec = lambda a: a.reshape(-1)
    fn = pl.pallas_call(
        _c2_body,
        grid=(nblk,),
        in_specs=[
            pl.BlockSpec((NB, CH, _LB), lambda j: (0, 0, j)),
            pl.BlockSpec((NB * CH,), lambda j: (0,)),
            pl.BlockSpec((NB * MLP_HID,), lambda j: (0,)),
            pl.BlockSpec((NB * MLP_HID,), lambda j: (0,)),
            pl.BlockSpec((NB * MLP_HID,), lambda j: (0,)),
            pl.BlockSpec((NB * MLP_HID,), lambda j: (0,)),
            pl.BlockSpec((NB * 3,), lambda j: (0,)),
            pl.BlockSpec((NB * MLP_HID, NB * CH), lambda j: (0, 0)),
            pl.BlockSpec((NB * MLP_HID,), lambda j: (0,)),
            pl.BlockSpec((NB * MLP_HID, NB * MLP_HID), lambda j: (0, 0)),
            pl.BlockSpec((NB * MLP_HID,), lambda j: (0,)),
            pl.BlockSpec((NB * 3, NB * MLP_HID), lambda j: (0, 0)),
            pl.BlockSpec((NB * 3,), lambda j: (0,)),
        ],
        out_specs=pl.BlockSpec((NB * 3, _LB), lambda j: (0, j)),
        out_shape=jax.ShapeDtypeStruct((NB * 3, IMG * IMG), jnp.float32),
    )
    return fn(featv, vec(sc0), vec(dm0), vec(sc1), vec(dm1), vec(sc2),
              vec(dm2), w0b, jnp.tile(b0, NB), w1b, jnp.tile(b1, NB),
              w2b, jnp.tile(b2, NB))


def kernel(z, map_w0, map_b0, map_w1, map_b1, map_w2, map_b2, base_table,
           up_w0, up_a0, up_w1, up_a1, up_w2, up_a2, up_w3, up_a3,
           up_w4, up_a4, up_w5, up_a5, up_w6, up_a6,
           ml_a0, ml_w0, ml_b0, ml_a1, ml_w1, ml_b1, ml_a2, ml_w2, ml_b2):
    tabs, sc0, dm0, sc1, dm1, sc2, dm2 = _stage_a(
        (z, map_w0, map_b0, map_w1, map_b1, map_w2, map_b2, base_table,
         up_w0, up_a0, up_w1, up_a1, up_w2, up_a2, up_w3, up_a3,
         up_w4, up_a4, up_w5, up_a5, up_w6, up_a6,
         ml_a0, ml_w0, ml_a1, ml_w1, ml_a2, ml_w2))
    gs = _stage_b(tabs.reshape(-1), jnp.asarray(_IDX_FLAT))
    pad3 = lambda l, w: jnp.pad(gs[l].reshape(8, _M[l], _R1[l]),
                                ((0, 0), (0, 0), (0, w - _R1[l])))
    ga = jnp.concatenate([pad3(l, 128) for l in _GA], axis=1)
    gb = jnp.concatenate([pad3(l, 256) for l in _GB], axis=1)
    feat = _stage_c1(ga, gb)
    featv = feat.reshape(NB, CH, IMG * IMG)
    o = _stage_c2(featv, sc0, dm0, sc1, dm1, sc2, dm2,
                  ml_w0, ml_b0, ml_w1, ml_b1, ml_w2, ml_b2)
    return o.reshape(NB, 3, IMG, IMG)


# speedup vs baseline: 1.1003x; 1.1003x over previous
import functools
import math

import jax
import jax.numpy as jnp
import numpy as np
from jax import lax
from jax.experimental import pallas as pl
from jax.experimental.pallas import tpu as pltpu
from jax.experimental.pallas import tpu_sc as plsc

TABLE_NUM = 16
T = 4096
IMG = 256
NB = 4
CH = 32
RES_MIN, RES_MAX = 16, 256
MLP_HID = 64
_SQ2 = math.sqrt(2.0)

_growth = math.exp((math.log(RES_MAX) - math.log(RES_MIN)) / (TABLE_NUM - 1))
_RES = [int(math.floor(RES_MIN * (_growth ** l))) for l in range(TABLE_NUM)]
_R1 = [r + 1 for r in _RES]
_HASH_K = np.uint32(2654435761)


def _interp_mat(res):
    p = np.arange(IMG, dtype=np.float64)
    pos = (p + 0.5) * res / IMG
    p0 = np.floor(pos).astype(np.int64)
    f = pos - p0
    w = np.zeros((IMG, res + 1), np.float32)
    w[np.arange(IMG), p0] = (1.0 - f).astype(np.float32)
    w[np.arange(IMG), p0 + 1] += f.astype(np.float32)
    return w


_WY = [_interp_mat(r) for r in _RES]
_WXT = [np.ascontiguousarray(w.T) for w in _WY]


def _hash_idx(res):
    yi = np.arange(res + 1, dtype=np.uint32)
    xi = np.arange(res + 1, dtype=np.uint32)
    h = (xi[None, :] ^ (yi[:, None] * _HASH_K)) & np.uint32(T - 1)
    return h.astype(np.int32).reshape(-1)


_N = [r1 * r1 for r1 in _R1]
_CHK = 8192
_W = [((n + 127) // 128) * 128 for n in _N]
_NCH = [(w + _CHK - 1) // _CHK for w in _W]
_M = []
for _l in range(TABLE_NUM):
    _m = max(_R1[_l], -(-_W[_l] // _R1[_l]))
    while (_m * _R1[_l]) % 128 != 0:
        _m += 1
    _M.append(_m)
_SEG = [_M[_l] * _R1[_l] for _l in range(TABLE_NUM)]
_OFF = []
_segs = []
_off = 0
for _l in range(TABLE_NUM):
    _OFF.append(_off)
    _seg = np.zeros((_W[_l],), np.int32)
    _seg[: _N[_l]] = _hash_idx(_RES[_l])
    _segs.append(_seg)
    _off += _W[_l]
_IDX_FLAT = np.concatenate(_segs)

_JOBS = []
for _l in range(TABLE_NUM):
    for _ci in range(_NCH[_l]):
        _k = min(_CHK, _W[_l] - _ci * _CHK)
        _JOBS.append((_l, _ci, _k))
_NW = 32
_ASSIGN = [j % _NW for j in range(len(_JOBS))]


def _lrelu(y):
    return jnp.where(y >= 0, y, 0.2 * y) * _SQ2


def _dot_t(x, w):
    return lax.dot_general(x, w, (((1,), (1,)), ((), ())),
                           preferred_element_type=jnp.float32)


def _dot(x, w):
    return lax.dot_general(x, w, (((1,), (0,)), ((), ())),
                           preferred_element_type=jnp.float32)


def _dot_hi(x, w):
    return lax.dot_general(x, w, (((1,), (0,)), ((), ())),
                           precision=lax.Precision.HIGHEST,
                           preferred_element_type=jnp.float32)


_UP_FIN = [32 * (2 ** i) for i in range(7)]


def _a_body(z, mw0, mb0, mw1, mb2_0, mw2, mb2_2, bt,
            uw0, ua0, uw1, ua1, uw2, ua2, uw3, ua3, uw4, ua4, uw5, ua5,
            uw6, ua6, mla0, mlw0, mla1, mlw1, mla2, mlw2,
            tabs_o, sc0_o, dm0_o, sc1_o, dm1_o, sc2_o, dm2_o):
    zv = z[...]
    s = _lrelu(_dot_t(zv, mw0[...] * (1.0 / math.sqrt(512.0))) + mb0[...][None, :])
    s = _lrelu(_dot_t(s, mw1[...] * (1.0 / math.sqrt(256.0))) + mb2_0[...][None, :])
    s = _lrelu(_dot_t(s, mw2[...] * (1.0 / math.sqrt(256.0))) + mb2_2[...][None, :])

    x = jnp.broadcast_to(bt[...][None], (NB, CH, 32)).reshape(NB * CH, 32)
    uws = [uw0, uw1, uw2, uw3, uw4, uw5, uw6]
    uas = [ua0, ua1, ua2, ua3, ua4, ua5, ua6]
    for i in range(7):
        fin = _UP_FIN[i]
        w = uws[i][...]
        scale = _dot_t(s, uas[i][...]) + 1.0
        ssq = scale * scale
        acc = jnp.zeros((NB, 2 * fin), jnp.float32)
        ckn = 256
        for c0 in range(0, fin, ckn):
            ck = min(ckn, fin - c0)
            wz = w[:, c0:c0 + ck]
            acc = acc + _dot_t(ssq[:, c0:c0 + ck], wz * wz)
        demod = lax.rsqrt(acc + 1e-8)
        xs = x * jnp.broadcast_to(scale[:, None, :], (NB, CH, fin)).reshape(NB * CH, fin)
        y = _dot_t(xs, w)
        y = y * jnp.broadcast_to(demod[:, None, :], (NB, CH, 2 * fin)).reshape(NB * CH, 2 * fin)
        x = _lrelu(y)
    tabs_o[...] = x.reshape(NB, CH, T)

    def modpair(a_ref, w_ref):
        sc = _dot_t(s, a_ref[...]) + 1.0
        wv = w_ref[...]
        dm = lax.rsqrt(_dot_t(sc * sc, wv * wv) + 1e-8)
        return sc, dm

    sc0, dm0 = modpair(mla0, mlw0)
    sc1, dm1 = modpair(mla1, mlw1)
    sc2, dm2 = modpair(mla2, mlw2)
    sc0_o[...] = sc0
    dm0_o[...] = dm0
    sc1_o[...] = sc1
    dm1_o[...] = dm1
    sc2_o[...] = sc2
    dm2_o[...] = dm2


def _stage_a(args):
    out_shape = (
        jax.ShapeDtypeStruct((NB, CH, T), jnp.float32),
        jax.ShapeDtypeStruct((NB, CH), jnp.float32),
        jax.ShapeDtypeStruct((NB, MLP_HID), jnp.float32),
        jax.ShapeDtypeStruct((NB, MLP_HID), jnp.float32),
        jax.ShapeDtypeStruct((NB, MLP_HID), jnp.float32),
        jax.ShapeDtypeStruct((NB, MLP_HID), jnp.float32),
        jax.ShapeDtypeStruct((NB, 3), jnp.float32),
    )
    return pl.pallas_call(_a_body, out_shape=out_shape)(*args)


def _sc_body(tabs, idx_hbm, *refs):
    outs = refs[:TABLE_NUM]
    tab_v = refs[TABLE_NUM:TABLE_NUM + 8]
    idx_v = refs[TABLE_NUM + 8]
    gbuf = refs[TABLE_NUM + 9]
    wid = lax.axis_index("s") * 2 + lax.axis_index("c")
    for j, (l, ci, kk) in enumerate(_JOBS):
        @pl.when(wid == _ASSIGN[j])
        def _(l=l, ci=ci, kk=kk):
            for b in range(NB):
                for c in range(2):
                    row = b * CH + 2 * l + c
                    pltpu.sync_copy(tabs.at[pl.ds(row * T, T)], tab_v[2 * b + c])
            pltpu.sync_copy(idx_hbm.at[pl.ds(_OFF[l] + ci * _CHK, kk)],
                            idx_v.at[pl.ds(0, kk)])

            def body(i, carry):
                iv0 = idx_v[pl.ds(i * 32, 16)]
                iv1 = idx_v[pl.ds(i * 32 + 16, 16)]
                for ch in range(8):
                    g0 = plsc.load_gather(tab_v[ch], [iv0])
                    g1 = plsc.load_gather(tab_v[ch], [iv1])
                    gbuf[pl.ds(ch * kk + i * 32, 16)] = g0
                    gbuf[pl.ds(ch * kk + i * 32 + 16, 16)] = g1
                return carry

            lax.fori_loop(0, kk // 32, body, 0)
            for ch in range(8):
                pltpu.sync_copy(
                    gbuf.at[pl.ds(ch * kk, kk)],
                    outs[l].at[pl.ds(ch * _SEG[l] + ci * _CHK, kk)])


def _stage_b(tabs, idx):
    fn = pl.kernel(
        _sc_body,
        out_type=[jax.ShapeDtypeStruct((8 * _SEG[l],), jnp.float32)
                  for l in range(TABLE_NUM)],
        mesh=plsc.VectorSubcoreMesh(core_axis_name="c", subcore_axis_name="s"),
        compiler_params=pltpu.CompilerParams(needs_layout_passes=False),
        scratch_types=(
            [pltpu.VMEM((T,), jnp.float32) for _ in range(8)]
            + [pltpu.VMEM((_CHK,), jnp.int32)]
            + [pltpu.VMEM((8 * _CHK,), jnp.float32)]
        ),
    )
    return fn(tabs, idx)


def _c1_body(*refs):
    gs = refs[0:TABLE_NUM]
    wys = refs[TABLE_NUM:2 * TABLE_NUM]
    wxts = refs[2 * TABLE_NUM:3 * TABLE_NUM]
    out = refs[3 * TABLE_NUM]
    for l in range(TABLE_NUM):
        wy = wys[l][...]
        wxt = wxts[l][...]
        for c in range(2):
            g = gs[l][c, 0:_R1[l], :]
            h = _dot_hi(g, wxt)
            f = _dot_hi(wy, h)
            out[0, 2 * l + c] = f


def _stage_c1(g3):
    in_specs = (
        [pl.BlockSpec((2, _M[l], _R1[l]), lambda b: (b, 0, 0))
         for l in range(TABLE_NUM)]
        + [pl.BlockSpec((IMG, _R1[l]), lambda b: (0, 0))
           for l in range(TABLE_NUM)]
        + [pl.BlockSpec((_R1[l], IMG), lambda b: (0, 0))
           for l in range(TABLE_NUM)]
    )
    fn = pl.pallas_call(
        _c1_body,
        grid=(NB,),
        in_specs=in_specs,
        out_specs=pl.BlockSpec((1, CH, IMG, IMG), lambda b: (b, 0, 0, 0)),
        out_shape=jax.ShapeDtypeStruct((NB, CH, IMG, IMG), jnp.float32),
    )
    return fn(*g3, *[jnp.asarray(w) for w in _WY],
              *[jnp.asarray(w) for w in _WXT])


_LB = 8192


def _c2_body(feat, sc0, dm0, sc1, dm1, sc2, dm2, w0, b0, w1, b1, w2, b2, out):
    x = feat[...].reshape(NB * CH, _LB)
    xs = x * sc0[...][:, None]
    h = _dot(w0[...], xs) * dm0[...][:, None] + b0[...][:, None]
    h = _lrelu(h)
    hs = h * sc1[...][:, None]
    h = _dot(w1[...], hs) * dm1[...][:, None] + b1[...][:, None]
    h = _lrelu(h)
    hs = h * sc2[...][:, None]
    out[...] = _dot(w2[...], hs) * dm2[...][:, None] + b2[...][:, None]


def _stage_c2(featv, sc0, dm0, sc1, dm1, sc2, dm2, w0, b0, w1, b1, w2, b2):
    nblk = IMG * IMG // _LB
    eye = jnp.eye(NB, dtype=jnp.float32)
    w0b = jnp.kron(eye, w0)
    w1b = jnp.kron(eye, w1)
    w2b = jnp.kron(eye, w2)
    vec = lambda a: a.reshape(-1)
    fn = pl.pallas_call(
        _c2_body,
        grid=(nblk,),
        in_specs=[
            pl.BlockSpec((NB, CH, _LB), lambda j: (0, 0, j)),
            pl.BlockSpec((NB * CH,), lambda j: (0,)),
            pl.BlockSpec((NB * MLP_HID,), lambda j: (0,)),
            pl.BlockSpec((NB * MLP_HID,), lambda j: (0,)),
            pl.BlockSpec((NB * MLP_HID,), lambda j: (0,)),
            pl.BlockSpec((NB * MLP_HID,), lambda j: (0,)),
            pl.BlockSpec((NB * 3,), lambda j: (0,)),
            pl.BlockSpec((NB * MLP_HID, NB * CH), lambda j: (0, 0)),
            pl.BlockSpec((NB * MLP_HID,), lambda j: (0,)),
            pl.BlockSpec((NB * MLP_HID, NB * MLP_HID), lambda j: (0, 0)),
            pl.BlockSpec((NB * MLP_HID,), lambda j: (0,)),
            pl.BlockSpec((NB * 3, NB * MLP_HID), lambda j: (0, 0)),
            pl.BlockSpec((NB * 3,), lambda j: (0,)),
        ],
        out_specs=pl.BlockSpec((NB * 3, _LB), lambda j: (0, j)),
        out_shape=jax.ShapeDtypeStruct((NB * 3, IMG * IMG), jnp.float32),
    )
    return fn(featv, vec(sc0), vec(dm0), vec(sc1), vec(dm1), vec(sc2),
              vec(dm2), w0b, jnp.tile(b0, NB), w1b, jnp.tile(b1, NB),
              w2b, jnp.tile(b2, NB))


def kernel(z, map_w0, map_b0, map_w1, map_b1, map_w2, map_b2, base_table,
           up_w0, up_a0, up_w1, up_a1, up_w2, up_a2, up_w3, up_a3,
           up_w4, up_a4, up_w5, up_a5, up_w6, up_a6,
           ml_a0, ml_w0, ml_b0, ml_a1, ml_w1, ml_b1, ml_a2, ml_w2, ml_b2):
    tabs, sc0, dm0, sc1, dm1, sc2, dm2 = _stage_a(
        (z, map_w0, map_b0, map_w1, map_b1, map_w2, map_b2, base_table,
         up_w0, up_a0, up_w1, up_a1, up_w2, up_a2, up_w3, up_a3,
         up_w4, up_a4, up_w5, up_a5, up_w6, up_a6,
         ml_a0, ml_w0, ml_a1, ml_w1, ml_a2, ml_w2))
    gs = _stage_b(tabs.reshape(-1), jnp.asarray(_IDX_FLAT))
    g3 = [gs[l].reshape(8, _M[l], _R1[l]) for l in range(TABLE_NUM)]
    feat = _stage_c1(g3)
    featv = feat.reshape(NB, CH, IMG * IMG)
    o = _stage_c2(featv, sc0, dm0, sc1, dm1, sc2, dm2,
                  ml_w0, ml_b0, ml_w1, ml_b1, ml_w2, ml_b2)
    return o.reshape(NB, 3, IMG, IMG)
